# Initial kernel scaffold; baseline (speedup 1.0000x reference)
import functools

import jax
import jax.numpy as jnp
from jax import lax
from jax.experimental import pallas as pl
from jax.experimental.pallas import tpu as pltpu
from jax.experimental.pallas import tpu_sc as plsc

NC = 2
NS = 16
K = 128
LANES = 16


def _mm_body(x_ref, w_ref, o_ref):
    o_ref[...] = jnp.dot(x_ref[...], w_ref[...],
                         preferred_element_type=jnp.float32)


def _combine_body(p_ref, b_ref, o_ref):
    o_ref[...] = p_ref[0] + p_ref[1] + b_ref[...]


def _sc_body(ch, n_rows, xw, srcg, dstg, valsg, out,
             src_v, dst_v, vals_v, rows, zbuf, acc, gsem, ssem):
    c = lax.axis_index("c")
    s = lax.axis_index("s")

    pltpu.sync_copy(srcg.at[c, s], src_v)
    pltpu.sync_copy(dstg.at[c, s], dst_v)
    pltpu.sync_copy(valsg.at[c, s], vals_v)

    zrows = zbuf.shape[0]
    per_tile = n_rows // NS
    reps = per_tile // zrows
    zero16 = jnp.zeros((LANES,), jnp.float32)

    def zrow(i, _):
        for cb in range(8):
            zbuf[i, pl.ds(cb * LANES, LANES)] = zero16
        return 0

    lax.fori_loop(0, zrows, zrow, 0)
    for kk in range(reps):
        pltpu.sync_copy(zbuf, acc.at[pl.ds(s * per_tile + kk * zrows, zrows)])
    plsc.subcore_barrier()

    def gather_desc(j, slot):
        return pltpu.make_async_copy(
            xw.at[src_v.at[j]], rows.at[pl.ds(slot * K, K)], gsem.at[slot])

    def scatter_desc(j, slot):
        return pltpu.make_async_copy(
            rows.at[pl.ds(slot * K, K)], acc.at[dst_v.at[j]], ssem.at[slot])

    pltpu.async_copy(xw.at[src_v.at[0]], rows.at[pl.ds(0, K)], gsem.at[0])

    def jbody(j, _):
        slot = lax.rem(j, 2)
        other = 1 - slot

        @pl.when(j >= 1)
        def _():
            scatter_desc(j - 1, other).wait()

        @pl.when(j + 1 < ch)
        def _():
            pltpu.async_copy(xw.at[src_v.at[j + 1]],
                             rows.at[pl.ds(other * K, K)], gsem.at[other])

        gather_desc(j, slot).wait()

        def gbody(g, _):
            v16 = vals_v[j, pl.ds(g * LANES, LANES)]
            for r in range(LANES):
                bc = jnp.take(v16, jnp.full((LANES,), r, jnp.int32),
                              mode="promise_in_bounds")
                e = slot * K + g * LANES + r
                for cb in range(8):
                    sl = pl.ds(cb * LANES, LANES)
                    rows[e, sl] = rows[e, sl] * bc
            return 0

        lax.fori_loop(0, K // LANES, gbody, 0)

        pltpu.async_copy(rows.at[pl.ds(slot * K, K)],
                         acc.at[dst_v.at[j]], ssem.at[slot], add=True)
        return 0

    lax.fori_loop(0, ch, jbody, 0)
    scatter_desc(ch - 1, lax.rem(ch - 1, 2)).wait()
    plsc.subcore_barrier()

    pltpu.sync_copy(acc.at[pl.ds(s * per_tile, per_tile)],
                    out.at[c, pl.ds(s * per_tile, per_tile)])


def kernel(input, edge_index, adj_values, W, b):
    n, d_in = input.shape
    d_out = W.shape[1]
    e = adj_values.shape[0]
    nw = NC * NS

    src = edge_index[1].astype(jnp.int32)
    dst = edge_index[0].astype(jnp.int32)
    vals = adj_values.astype(jnp.float32)
    ept = -(-e // nw)
    ch = -(-ept // K)
    e_pad = nw * ch * K
    pad = e_pad - e
    src = jnp.pad(src, (0, pad)).reshape(NC, NS, ch, K)
    dst = jnp.pad(dst, (0, pad)).reshape(NC, NS, ch, K)
    vals = jnp.pad(vals, (0, pad)).reshape(NC, NS, ch, K)

    rb = 1000
    xw = pl.pallas_call(
        _mm_body,
        grid=(n // rb,),
        in_specs=[
            pl.BlockSpec((rb, d_in), lambda i: (i, 0)),
            pl.BlockSpec((d_in, d_out), lambda i: (0, 0)),
        ],
        out_specs=pl.BlockSpec((rb, d_out), lambda i: (i, 0)),
        out_shape=jax.ShapeDtypeStruct((n, d_out), jnp.float32),
    )(input, W)

    mesh = plsc.VectorSubcoreMesh(core_axis_name="c", subcore_axis_name="s")
    sc = pl.kernel(
        functools.partial(_sc_body, ch, n),
        out_type=jax.ShapeDtypeStruct((NC, n, d_out), jnp.float32),
        mesh=mesh,
        scratch_types=[
            pltpu.VMEM((ch, K), jnp.int32),
            pltpu.VMEM((ch, K), jnp.int32),
            pltpu.VMEM((ch, K), jnp.float32),
            pltpu.VMEM((2 * K, d_out), jnp.float32),
            pltpu.VMEM((125, d_out), jnp.float32),
            pltpu.VMEM_SHARED((n, d_out), jnp.float32),
            pltpu.SemaphoreType.DMA((2,)),
            pltpu.SemaphoreType.DMA((2,)),
        ],
    )
    partials = sc(xw, src, dst, vals)

    out = pl.pallas_call(
        _combine_body,
        grid=(n // rb,),
        in_specs=[
            pl.BlockSpec((NC, rb, d_out), lambda i: (0, i, 0)),
            pl.BlockSpec((1, d_out), lambda i: (0, 0)),
        ],
        out_specs=pl.BlockSpec((rb, d_out), lambda i: (i, 0)),
        out_shape=jax.ShapeDtypeStruct((n, d_out), jnp.float32),
    )(partials, b.reshape(1, d_out))
    return out


# speedup vs baseline: 3.3311x; 3.3311x over previous
import functools

import jax
import jax.numpy as jnp
from jax import lax
from jax.experimental import pallas as pl
from jax.experimental.pallas import tpu as pltpu
from jax.experimental.pallas import tpu_sc as plsc

NC = 2
NS = 16
K = 128
G = 8
LANES = 16


def _bcast_lane(v16, lane):
    idx = jnp.full((LANES, 1), lane, jnp.int32)
    dnums = lax.GatherDimensionNumbers(
        offset_dims=(), collapsed_slice_dims=(0,), start_index_map=(0,))
    return lax.gather(v16, idx, dnums, slice_sizes=(1,),
                      mode=lax.GatherScatterMode.PROMISE_IN_BOUNDS)


def _mm_body(x_ref, w_ref, o_ref):
    o_ref[...] = jnp.dot(x_ref[...], w_ref[...],
                         preferred_element_type=jnp.float32)


def _combine_body(p_ref, b_ref, o_ref):
    o_ref[...] = p_ref[0] + p_ref[1] + b_ref[...]


def _sc_body(ch, n_pad, xw, srcg, dstg, valsg, zeros, out,
             src_r, dst_r, vals_r, rows, acc, gsem, ssem, isem):
    c = lax.axis_index("c")
    s = lax.axis_index("s")
    ng = ch // G
    per_tile = n_pad // NS

    pltpu.sync_copy(zeros.at[pl.ds(s * per_tile, per_tile)],
                    acc.at[pl.ds(s * per_tile, per_tile)])

    pltpu.sync_copy(srcg.at[c, s, 0], src_r.at[0])
    pltpu.sync_copy(dstg.at[c, s, 0], dst_r.at[0])
    pltpu.sync_copy(valsg.at[c, s, 0], vals_r.at[0])
    plsc.subcore_barrier()

    pltpu.async_copy(xw.at[src_r.at[0, 0]], rows.at[pl.ds(0, K)], gsem.at[0])

    def group_body(g, _):
        cur = lax.rem(g, 2)
        nxt = 1 - cur

        def jbody(jj, _):
            j = g * G + jj
            slot = lax.rem(j, 2)
            other = 1 - slot

            @pl.when(j >= 1)
            def _():
                jm = j - 1
                rs = lax.rem(jm // G, 2)
                pltpu.make_async_copy(
                    rows.at[pl.ds(other * K, K)],
                    acc.at[dst_r.at[rs, lax.rem(jm, G)]],
                    ssem.at[other]).wait()

            @pl.when(jnp.logical_and(jj == 0, g + 1 < ng))
            def _():
                pltpu.async_copy(srcg.at[c, s, g + 1], src_r.at[nxt],
                                 isem.at[nxt])
                pltpu.async_copy(dstg.at[c, s, g + 1], dst_r.at[nxt],
                                 isem.at[nxt])
                pltpu.async_copy(valsg.at[c, s, g + 1], vals_r.at[nxt],
                                 isem.at[nxt])

            @pl.when(jj + 1 < G)
            def _():
                pltpu.async_copy(xw.at[src_r.at[cur, jj + 1]],
                                 rows.at[pl.ds(other * K, K)], gsem.at[other])

            @pl.when(jnp.logical_and(jj + 1 == G, g + 1 < ng))
            def _():
                pltpu.make_async_copy(srcg.at[c, s, g + 1], src_r.at[nxt],
                                      isem.at[nxt]).wait()
                pltpu.make_async_copy(dstg.at[c, s, g + 1], dst_r.at[nxt],
                                      isem.at[nxt]).wait()
                pltpu.make_async_copy(valsg.at[c, s, g + 1], vals_r.at[nxt],
                                      isem.at[nxt]).wait()
                pltpu.async_copy(xw.at[src_r.at[nxt, 0]],
                                 rows.at[pl.ds(other * K, K)], gsem.at[other])

            pltpu.make_async_copy(xw.at[src_r.at[cur, jj]],
                                  rows.at[pl.ds(slot * K, K)],
                                  gsem.at[slot]).wait()

            def sbody(q, _):
                v16 = vals_r[cur, jj, pl.ds(q * LANES, LANES)]
                for r in range(LANES):
                    bc = _bcast_lane(v16, r)
                    e = slot * K + q * LANES + r
                    for cb in range(8):
                        sl = pl.ds(cb * LANES, LANES)
                        rows[e, sl] = rows[e, sl] * bc
                return 0

            lax.fori_loop(0, K // LANES, sbody, 0)

            pltpu.async_copy(rows.at[pl.ds(slot * K, K)],
                             acc.at[dst_r.at[cur, jj]], ssem.at[slot],
                             add=True)
            return 0

        lax.fori_loop(0, G, jbody, 0)
        return 0

    lax.fori_loop(0, ng, group_body, 0)

    jm = ch - 1
    pltpu.make_async_copy(
        rows.at[pl.ds((jm % 2) * K, K)],
        acc.at[dst_r.at[(jm // G) % 2, jm % G]],
        ssem.at[jm % 2]).wait()
    plsc.subcore_barrier()

    pltpu.sync_copy(acc.at[pl.ds(s * per_tile, per_tile)],
                    out.at[c, pl.ds(s * per_tile, per_tile)])


def kernel(input, edge_index, adj_values, W, b):
    n, d_in = input.shape
    d_out = W.shape[1]
    e = adj_values.shape[0]
    nw = NC * NS

    src = edge_index[1].astype(jnp.int32)
    dst = edge_index[0].astype(jnp.int32)
    vals = adj_values.astype(jnp.float32)
    ept = -(-e // nw)
    ch = -(-ept // (K * G)) * G
    e_pad = nw * ch * K
    pad = e_pad - e
    src = jnp.pad(src, (0, pad)).reshape(NC, NS, ch // G, G, K)
    dst = jnp.pad(dst, (0, pad)).reshape(NC, NS, ch // G, G, K)
    vals = jnp.pad(vals, (0, pad)).reshape(NC, NS, ch // G, G, K)

    rb = 1000
    xw = pl.pallas_call(
        _mm_body,
        grid=(n // rb,),
        in_specs=[
            pl.BlockSpec((rb, d_in), lambda i: (i, 0)),
            pl.BlockSpec((d_in, d_out), lambda i: (0, 0)),
        ],
        out_specs=pl.BlockSpec((rb, d_out), lambda i: (i, 0)),
        out_shape=jax.ShapeDtypeStruct((n, d_out), jnp.float32),
    )(input, W)

    n_pad = -(-n // (NS * K)) * (NS * K)
    zeros = jnp.zeros((n_pad, d_out), jnp.float32)
    mesh = plsc.VectorSubcoreMesh(core_axis_name="c", subcore_axis_name="s")
    sc = pl.kernel(
        functools.partial(_sc_body, ch, n_pad),
        out_type=jax.ShapeDtypeStruct((NC, n_pad, d_out), jnp.float32),
        mesh=mesh,
        scratch_types=[
            pltpu.VMEM((2, G, K), jnp.int32),
            pltpu.VMEM((2, G, K), jnp.int32),
            pltpu.VMEM((2, G, K), jnp.float32),
            pltpu.VMEM((2 * K, d_out), jnp.float32),
            pltpu.VMEM_SHARED((n_pad, d_out), jnp.float32),
            pltpu.SemaphoreType.DMA((2,)),
            pltpu.SemaphoreType.DMA((2,)),
            pltpu.SemaphoreType.DMA((2,)),
        ],
    )
    partials = sc(xw, src, dst, vals, zeros)

    cb = 1024
    out_pad = pl.pallas_call(
        _combine_body,
        grid=(n_pad // cb,),
        in_specs=[
            pl.BlockSpec((NC, cb, d_out), lambda i: (0, i, 0)),
            pl.BlockSpec((1, d_out), lambda i: (0, 0)),
        ],
        out_specs=pl.BlockSpec((cb, d_out), lambda i: (i, 0)),
        out_shape=jax.ShapeDtypeStruct((n_pad, d_out), jnp.float32),
    )(partials, b.reshape(1, d_out))
    return out_pad[:n]


# speedup vs baseline: 4.0531x; 1.2168x over previous
import functools

import jax
import jax.numpy as jnp
from jax import lax
from jax.experimental import pallas as pl
from jax.experimental.pallas import tpu as pltpu
from jax.experimental.pallas import tpu_sc as plsc

NC = 2
NS = 16
K = 128
G = 8
SS = 4
SR = K // SS
LANES = 16


def _bcast_lane(v16, lane):
    idx = jnp.full((LANES, 1), lane, jnp.int32)
    dnums = lax.GatherDimensionNumbers(
        offset_dims=(), collapsed_slice_dims=(0,), start_index_map=(0,))
    return lax.gather(v16, idx, dnums, slice_sizes=(1,),
                      mode=lax.GatherScatterMode.PROMISE_IN_BOUNDS)


def _mm_body(x_ref, w_ref, o_ref):
    o_ref[...] = jnp.dot(x_ref[...], w_ref[...],
                         preferred_element_type=jnp.float32)


def _combine_body(p_ref, b_ref, o_ref):
    o_ref[...] = p_ref[0] + p_ref[1] + b_ref[...]


def _sc_body(ch, n_pad, xw, srcg, dstg, valsg, zeros, out,
             src_r, dst_r, vals_r, rows, acc, gsem, ssem, isem):
    c = lax.axis_index("c")
    s = lax.axis_index("s")
    ng = ch // G
    per_tile = n_pad // NS

    pltpu.sync_copy(zeros.at[pl.ds(s * per_tile, per_tile)],
                    acc.at[pl.ds(s * per_tile, per_tile)])

    pltpu.sync_copy(srcg.at[c, s, 0], src_r.at[0])
    pltpu.sync_copy(dstg.at[c, s, 0], dst_r.at[0])
    pltpu.sync_copy(valsg.at[c, s, 0], vals_r.at[0])
    plsc.subcore_barrier()

    def issue_gather(rs, jj, slot):
        for q in range(SS):
            pltpu.async_copy(
                xw.at[src_r.at[rs, jj, pl.ds(q * SR, SR)]],
                rows.at[pl.ds(slot * K + q * SR, SR)], gsem.at[slot])

    def wait_gather(rs, jj, slot):
        for q in range(SS):
            pltpu.make_async_copy(
                xw.at[src_r.at[rs, jj, pl.ds(q * SR, SR)]],
                rows.at[pl.ds(slot * K + q * SR, SR)], gsem.at[slot]).wait()

    issue_gather(0, 0, 0)

    def group_body(g, _):
        cur = lax.rem(g, 2)
        nxt = 1 - cur

        def jbody(jj, _):
            j = g * G + jj
            slot = lax.rem(j, 2)
            other = 1 - slot

            @pl.when(j >= 1)
            def _():
                jm = j - 1
                rs = lax.rem(jm // G, 2)
                pltpu.make_async_copy(
                    rows.at[pl.ds(other * K, K)],
                    acc.at[dst_r.at[rs, lax.rem(jm, G)]],
                    ssem.at[other]).wait()

            @pl.when(jnp.logical_and(jj == 0, g + 1 < ng))
            def _():
                pltpu.async_copy(srcg.at[c, s, g + 1], src_r.at[nxt],
                                 isem.at[nxt])
                pltpu.async_copy(dstg.at[c, s, g + 1], dst_r.at[nxt],
                                 isem.at[nxt])
                pltpu.async_copy(valsg.at[c, s, g + 1], vals_r.at[nxt],
                                 isem.at[nxt])

            @pl.when(jj + 1 < G)
            def _():
                issue_gather(cur, jj + 1, other)

            @pl.when(jnp.logical_and(jj + 1 == G, g + 1 < ng))
            def _():
                pltpu.make_async_copy(srcg.at[c, s, g + 1], src_r.at[nxt],
                                      isem.at[nxt]).wait()
                pltpu.make_async_copy(dstg.at[c, s, g + 1], dst_r.at[nxt],
                                      isem.at[nxt]).wait()
                pltpu.make_async_copy(valsg.at[c, s, g + 1], vals_r.at[nxt],
                                      isem.at[nxt]).wait()
                issue_gather(nxt, 0, other)

            wait_gather(cur, jj, slot)

            def sbody(q, _):
                v16 = vals_r[cur, jj, pl.ds(q * LANES, LANES)]
                for r in range(LANES):
                    bc = _bcast_lane(v16, r)
                    e = slot * K + q * LANES + r
                    for cb in range(8):
                        sl = pl.ds(cb * LANES, LANES)
                        rows[e, sl] = rows[e, sl] * bc
                return 0

            lax.fori_loop(0, K // LANES, sbody, 0)

            pltpu.async_copy(rows.at[pl.ds(slot * K, K)],
                             acc.at[dst_r.at[cur, jj]], ssem.at[slot],
                             add=True)
            return 0

        lax.fori_loop(0, G, jbody, 0)
        return 0

    lax.fori_loop(0, ng, group_body, 0)

    jm = ch - 1
    pltpu.make_async_copy(
        rows.at[pl.ds((jm % 2) * K, K)],
        acc.at[dst_r.at[(jm // G) % 2, jm % G]],
        ssem.at[jm % 2]).wait()
    plsc.subcore_barrier()

    pltpu.sync_copy(acc.at[pl.ds(s * per_tile, per_tile)],
                    out.at[c, pl.ds(s * per_tile, per_tile)])


def kernel(input, edge_index, adj_values, W, b):
    n, d_in = input.shape
    d_out = W.shape[1]
    e = adj_values.shape[0]
    nw = NC * NS

    src = edge_index[1].astype(jnp.int32)
    dst = edge_index[0].astype(jnp.int32)
    vals = adj_values.astype(jnp.float32)
    ept = -(-e // nw)
    ch = -(-ept // (K * G)) * G
    e_pad = nw * ch * K
    pad = e_pad - e
    pad_idx = jnp.arange(pad, dtype=jnp.int32) % n
    src = jnp.concatenate([src, pad_idx]).reshape(NC, NS, ch // G, G, K)
    dst = jnp.concatenate([dst, pad_idx]).reshape(NC, NS, ch // G, G, K)
    vals = jnp.concatenate([vals, jnp.zeros((pad,), jnp.float32)])
    vals = vals.reshape(NC, NS, ch // G, G, K)

    rb = 1000
    xw = pl.pallas_call(
        _mm_body,
        grid=(n // rb,),
        in_specs=[
            pl.BlockSpec((rb, d_in), lambda i: (i, 0)),
            pl.BlockSpec((d_in, d_out), lambda i: (0, 0)),
        ],
        out_specs=pl.BlockSpec((rb, d_out), lambda i: (i, 0)),
        out_shape=jax.ShapeDtypeStruct((n, d_out), jnp.float32),
    )(input, W)

    n_pad = -(-n // (NS * K)) * (NS * K)
    zeros = jnp.zeros((n_pad, d_out), jnp.float32)
    mesh = plsc.VectorSubcoreMesh(core_axis_name="c", subcore_axis_name="s")
    sc = pl.kernel(
        functools.partial(_sc_body, ch, n_pad),
        out_type=jax.ShapeDtypeStruct((NC, n_pad, d_out), jnp.float32),
        mesh=mesh,
        scratch_types=[
            pltpu.VMEM((2, G, K), jnp.int32),
            pltpu.VMEM((2, G, K), jnp.int32),
            pltpu.VMEM((2, G, K), jnp.float32),
            pltpu.VMEM((2 * K, d_out), jnp.float32),
            pltpu.VMEM_SHARED((n_pad, d_out), jnp.float32),
            pltpu.SemaphoreType.DMA((2,)),
            pltpu.SemaphoreType.DMA((2,)),
            pltpu.SemaphoreType.DMA((2,)),
        ],
    )
    partials = sc(xw, src, dst, vals, zeros)

    cb = 1024
    out_pad = pl.pallas_call(
        _combine_body,
        grid=(n_pad // cb,),
        in_specs=[
            pl.BlockSpec((NC, cb, d_out), lambda i: (0, i, 0)),
            pl.BlockSpec((1, d_out), lambda i: (0, 0)),
        ],
        out_specs=pl.BlockSpec((cb, d_out), lambda i: (i, 0)),
        out_shape=jax.ShapeDtypeStruct((n_pad, d_out), jnp.float32),
    )(partials, b.reshape(1, d_out))
    return out_pad[:n]


# speedup vs baseline: 8.6947x; 2.1452x over previous
import functools

import jax
import jax.numpy as jnp
from jax import lax
from jax.experimental import pallas as pl
from jax.experimental.pallas import tpu as pltpu
from jax.experimental.pallas import tpu_sc as plsc

NC = 2
NS = 16
K = 128
G = 8
SS = 8
SR = K // SS
LANES = 16


def _bcast_lane(v16, lane):
    idx = jnp.full((LANES, 1), lane, jnp.int32)
    dnums = lax.GatherDimensionNumbers(
        offset_dims=(), collapsed_slice_dims=(0,), start_index_map=(0,))
    return lax.gather(v16, idx, dnums, slice_sizes=(1,),
                      mode=lax.GatherScatterMode.PROMISE_IN_BOUNDS)


def _mm_body(x_ref, w_ref, o_ref):
    o_ref[...] = jnp.dot(x_ref[...], w_ref[...],
                         preferred_element_type=jnp.float32)


def _combine_body(p_ref, b_ref, o_ref):
    o_ref[...] = p_ref[0] + p_ref[1] + b_ref[...]


def _sc_body(ch, n_pad, xw, srcg, dstg, valsg, zeros, out,
             src_r, dst_r, vals_r, rows, acc, gsem, ssem, isem):
    c = lax.axis_index("c")
    s = lax.axis_index("s")
    ng = ch // G
    per_tile = n_pad // NS

    pltpu.sync_copy(zeros.at[pl.ds(s * per_tile, per_tile)],
                    acc.at[pl.ds(s * per_tile, per_tile)])

    pltpu.sync_copy(srcg.at[c, s, 0], src_r.at[0])
    pltpu.sync_copy(dstg.at[c, s, 0], dst_r.at[0])
    pltpu.sync_copy(valsg.at[c, s, 0], vals_r.at[0])
    plsc.subcore_barrier()

    def issue_gather(rs, jj, slot):
        for q in range(SS):
            pltpu.async_copy(
                xw.at[src_r.at[rs, jj, pl.ds(q * SR, SR)]],
                rows.at[pl.ds(slot * K + q * SR, SR)], gsem.at[slot])

    def wait_gather(rs, jj, slot):
        for q in range(SS):
            pltpu.make_async_copy(
                xw.at[src_r.at[rs, jj, pl.ds(q * SR, SR)]],
                rows.at[pl.ds(slot * K + q * SR, SR)], gsem.at[slot]).wait()

    issue_gather(0, 0, 0)

    def group_body(g, _):
        cur = lax.rem(g, 2)
        nxt = 1 - cur

        def jbody(jj, _):
            j = g * G + jj
            slot = lax.rem(j, 2)
            other = 1 - slot

            @pl.when(j >= 1)
            def _():
                jm = j - 1
                rs = lax.rem(jm // G, 2)
                pltpu.make_async_copy(
                    rows.at[pl.ds(other * K, K)],
                    acc.at[dst_r.at[rs, lax.rem(jm, G)]],
                    ssem.at[other]).wait()

            @pl.when(jnp.logical_and(jj == 0, g + 1 < ng))
            def _():
                pltpu.async_copy(srcg.at[c, s, g + 1], src_r.at[nxt],
                                 isem.at[nxt])
                pltpu.async_copy(dstg.at[c, s, g + 1], dst_r.at[nxt],
                                 isem.at[nxt])
                pltpu.async_copy(valsg.at[c, s, g + 1], vals_r.at[nxt],
                                 isem.at[nxt])

            @pl.when(jj + 1 < G)
            def _():
                issue_gather(cur, jj + 1, other)

            @pl.when(jnp.logical_and(jj + 1 == G, g + 1 < ng))
            def _():
                pltpu.make_async_copy(srcg.at[c, s, g + 1], src_r.at[nxt],
                                      isem.at[nxt]).wait()
                pltpu.make_async_copy(dstg.at[c, s, g + 1], dst_r.at[nxt],
                                      isem.at[nxt]).wait()
                pltpu.make_async_copy(valsg.at[c, s, g + 1], vals_r.at[nxt],
                                      isem.at[nxt]).wait()
                issue_gather(nxt, 0, other)

            wait_gather(cur, jj, slot)

            def sbody(q, _):
                v16 = vals_r[cur, jj, pl.ds(q * LANES, LANES)]
                for r in range(LANES):
                    bc = _bcast_lane(v16, r)
                    e = slot * K + q * LANES + r
                    for cb in range(8):
                        sl = pl.ds(cb * LANES, LANES)
                        rows[e, sl] = rows[e, sl] * bc
                return 0

            lax.fori_loop(0, K // LANES, sbody, 0)

            pltpu.async_copy(rows.at[pl.ds(slot * K, K)],
                             acc.at[dst_r.at[cur, jj]], ssem.at[slot],
                             add=True)
            return 0

        lax.fori_loop(0, G, jbody, 0)
        return 0

    lax.fori_loop(0, ng, group_body, 0)

    jm = ch - 1
    pltpu.make_async_copy(
        rows.at[pl.ds((jm % 2) * K, K)],
        acc.at[dst_r.at[(jm // G) % 2, jm % G]],
        ssem.at[jm % 2]).wait()
    plsc.subcore_barrier()

    pltpu.sync_copy(acc.at[pl.ds(s * per_tile, per_tile)],
                    out.at[c, pl.ds(s * per_tile, per_tile)])


def kernel(input, edge_index, adj_values, W, b):
    n, d_in = input.shape
    d_out = W.shape[1]
    e = adj_values.shape[0]
    nw = NC * NS

    src = edge_index[1].astype(jnp.int32)
    dst = edge_index[0].astype(jnp.int32)
    vals = adj_values.astype(jnp.float32)
    ept = -(-e // nw)
    ch = -(-ept // (K * G)) * G
    e_pad = nw * ch * K
    pad = e_pad - e
    pad_idx = jnp.arange(pad, dtype=jnp.int32) % n
    src = jnp.concatenate([src, pad_idx]).reshape(NC, NS, ch // G, G, K)
    dst = jnp.concatenate([dst, pad_idx]).reshape(NC, NS, ch // G, G, K)
    vals = jnp.concatenate([vals, jnp.zeros((pad,), jnp.float32)])
    vals = vals.reshape(NC, NS, ch // G, G, K)

    rb = 1000
    xw = pl.pallas_call(
        _mm_body,
        grid=(n // rb,),
        in_specs=[
            pl.BlockSpec((rb, d_in), lambda i: (i, 0)),
            pl.BlockSpec((d_in, d_out), lambda i: (0, 0)),
        ],
        out_specs=pl.BlockSpec((rb, d_out), lambda i: (i, 0)),
        out_shape=jax.ShapeDtypeStruct((n, d_out), jnp.float32),
    )(input, W)

    n_pad = -(-n // (NS * K)) * (NS * K)
    zeros = jnp.zeros((n_pad, d_out), jnp.float32)
    mesh = plsc.VectorSubcoreMesh(core_axis_name="c", subcore_axis_name="s")
    sc = pl.kernel(
        functools.partial(_sc_body, ch, n_pad),
        out_type=jax.ShapeDtypeStruct((NC, n_pad, d_out), jnp.float32),
        mesh=mesh,
        scratch_types=[
            pltpu.VMEM((2, G, K), jnp.int32),
            pltpu.VMEM((2, G, K), jnp.int32),
            pltpu.VMEM((2, G, K), jnp.float32),
            pltpu.VMEM((2 * K, d_out), jnp.float32),
            pltpu.VMEM_SHARED((n_pad, d_out), jnp.float32),
            pltpu.SemaphoreType.DMA((2,)),
            pltpu.SemaphoreType.DMA((2,)),
            pltpu.SemaphoreType.DMA((2,)),
        ],
    )
    partials = sc(xw, src, dst, vals, zeros)

    cb = 1024
    out_pad = pl.pallas_call(
        _combine_body,
        grid=(n_pad // cb,),
        in_specs=[
            pl.BlockSpec((NC, cb, d_out), lambda i: (0, i, 0)),
            pl.BlockSpec((1, d_out), lambda i: (0, 0)),
        ],
        out_specs=pl.BlockSpec((cb, d_out), lambda i: (i, 0)),
        out_shape=jax.ShapeDtypeStruct((n_pad, d_out), jnp.float32),
    )(partials, b.reshape(1, d_out))
    return out_pad[:n]
